# baseline (speedup 1.0000x reference)
import jax
import jax.numpy as jnp
from jax.experimental import pallas as pl
from jax.experimental.pallas import tpu as pltpu

_TAU = 0.01
_SINKHORN_ITERS = 5
_EPS = 1e-05
_TOP_K = 5

_B, _N, _M = 8, 2048, 2048
_CHUNK = 256
_NCH = _N // _CHUNK


def _apml_kernel(pred_ref, gtt_ref, out_ref, s_ref, r_ref):
    b = pl.program_id(0)

    gtt = gtt_ref[0]
    b2 = jnp.sum(gtt * gtt, axis=0, keepdims=True)

    a = pred_ref[0]
    a2 = jnp.sum(a * a, axis=1, keepdims=True)
    ab = jnp.dot(a, gtt, preferred_element_type=jnp.float32)
    d2 = jnp.maximum(a2 + b2 - 2.0 * ab, 1e-12)
    dall = d2 * jax.lax.rsqrt(d2)
    s = jnp.exp(dall * (-1.0 / _TAU))
    s_ref[...] = s
    rs = jnp.sum(s, axis=1, keepdims=True)
    r_new = 1.0 / (rs + _EPS)
    r_ref[...] = r_new
    cs = jnp.sum(s * r_new, axis=0, keepdims=True)
    c0 = 1.0 / (cs + _EPS)

    def sink(_, c):
        rs = jnp.dot(s_ref[...], c.T, preferred_element_type=jnp.float32)
        r_old = r_ref[...]
        r_new = r_old / (r_old * rs + _EPS)
        r_ref[...] = r_new
        cs = jnp.dot(r_new.T, s_ref[...], preferred_element_type=jnp.float32)
        return c / (c * cs + _EPS)

    c = jax.lax.fori_loop(0, _SINKHORN_ITERS - 1, sink, c0, unroll=False)

    lnc = jnp.log(jnp.maximum(c, 1e-45))

    def stats(w, m, k, s1, s2):
        eq = w == m
        cnt = jnp.sum(jnp.where(eq, 1.0, 0.0), axis=1, keepdims=True)
        slnc = jnp.sum(jnp.where(eq, lnc, 0.0), axis=1, keepdims=True)
        live = (k < float(_TOP_K)) & (m > 0.0)
        sum_d = (cnt * jnp.log(m) - slnc) * (-_TAU)
        s1 = s1 + jnp.where(live, m * cnt, 0.0)
        s2 = s2 + jnp.where(live, m * sum_d, 0.0)
        k = k + jnp.where(k < float(_TOP_K), cnt, 0.0)
        return k, s1, s2

    z = jnp.zeros((_N, 1), jnp.float32)
    w1 = s_ref[...] * c
    s_ref[...] = w1
    m = jnp.max(w1, axis=1, keepdims=True)
    k, s1, s2 = stats(w1, m, z, z, z)

    def tk(_, carry):
        thresh, k, s1, s2 = carry
        w = s_ref[...]
        masked = jnp.where(w < thresh, w, -jnp.inf)
        m = jnp.max(masked, axis=1, keepdims=True)
        k, s1, s2 = stats(w, m, k, s1, s2)
        return m, k, s1, s2

    _, _, s1, s2 = jax.lax.fori_loop(0, _TOP_K - 1, tk, (m, k, s1, s2),
                                     unroll=True)
    rfin = r_ref[...]
    row_loss = (rfin * s2) / (rfin * s1 + _EPS)
    loss_b = jnp.sum(row_loss)

    @pl.when(b == 0)
    def _():
        out_ref[...] = jnp.zeros((1, 1), jnp.float32)

    out_ref[...] = out_ref[...] + loss_b * (1.0 / _B)


def _apml(pred, gt, interpret=False):
    predp = jnp.pad(pred, ((0, 0), (0, 0), (0, 5)))
    gttp = jnp.pad(jnp.swapaxes(gt, 1, 2), ((0, 0), (0, 5), (0, 0)))
    out = pl.pallas_call(
        _apml_kernel,
        grid=(_B,),
        in_specs=[
            pl.BlockSpec((1, _N, 8), lambda b: (b, 0, 0)),
            pl.BlockSpec((1, 8, _M), lambda b: (b, 0, 0)),
        ],
        out_specs=pl.BlockSpec((1, 1), lambda b: (0, 0)),
        out_shape=jax.ShapeDtypeStruct((1, 1), jnp.float32),
        scratch_shapes=[
            pltpu.VMEM((_N, _M), jnp.float32),
            pltpu.VMEM((_N, 1), jnp.float32),
        ],
        compiler_params=pltpu.CompilerParams(
            dimension_semantics=("arbitrary",),
        ),
        interpret=interpret,
    )(predp, gttp)
    return out[0, 0]


def kernel(pred, gt):
    return _apml(pred, gt)


# speedup vs baseline: 1.2052x; 1.2052x over previous
import jax
import jax.numpy as jnp
from jax.experimental import pallas as pl
from jax.experimental.pallas import tpu as pltpu

_TAU = 0.01
_SINKHORN_ITERS = 5
_EPS = 1e-05
_TOP_K = 5

_B, _N, _M = 8, 2048, 2048
_CHUNK = 256
_NCH = _N // _CHUNK


def _apml_kernel(pred_ref, gtt_ref, out_ref, s_ref, r_ref):
    b = pl.program_id(0)

    gtt = gtt_ref[0]
    b2 = jnp.sum(gtt * gtt, axis=0, keepdims=True)

    def build(i, colacc):
        sl = pl.ds(i * _CHUNK, _CHUNK)
        a = pred_ref[0, sl, :]
        a2 = jnp.sum(a * a, axis=1, keepdims=True)
        ab = jnp.dot(a, gtt, preferred_element_type=jnp.float32)
        d2 = jnp.maximum(a2 + b2 - 2.0 * ab, 1e-12)
        dchunk = d2 * jax.lax.rsqrt(d2)
        s = jnp.exp(dchunk * (-1.0 / _TAU))
        s_ref[sl, :] = s
        rs = jnp.sum(s, axis=1, keepdims=True)
        r_new = 1.0 / (rs + _EPS)
        r_ref[sl, :] = r_new
        return colacc + jnp.sum(s * r_new, axis=0, keepdims=True)

    cs = jax.lax.fori_loop(0, _NCH, build, jnp.zeros((1, _M), jnp.float32),
                           unroll=False)
    c0 = 1.0 / (cs + _EPS)

    def sink(_, c):
        rs = jnp.dot(s_ref[...], c.T, preferred_element_type=jnp.float32)
        r_old = r_ref[...]
        r_new = r_old / (r_old * rs + _EPS)
        r_ref[...] = r_new
        cs = jnp.dot(r_new.T, s_ref[...], preferred_element_type=jnp.float32)
        return c / (c * cs + _EPS)

    c = jax.lax.fori_loop(0, _SINKHORN_ITERS - 1, sink, c0, unroll=False)

    lnc = jnp.log(jnp.maximum(c, 1e-45))

    def stats(w, m, k, s1, s2):
        eq = w == m
        cnt = jnp.sum(jnp.where(eq, 1.0, 0.0), axis=1, keepdims=True)
        slnc = jnp.sum(jnp.where(eq, lnc, 0.0), axis=1, keepdims=True)
        live = (k < float(_TOP_K)) & (m > 0.0)
        sum_d = (cnt * jnp.log(m) - slnc) * (-_TAU)
        s1 = s1 + jnp.where(live, m * cnt, 0.0)
        s2 = s2 + jnp.where(live, m * sum_d, 0.0)
        k = k + jnp.where(k < float(_TOP_K), cnt, 0.0)
        return k, s1, s2

    z = jnp.zeros((_N, 1), jnp.float32)
    w1 = s_ref[...] * c
    s_ref[...] = w1
    m = jnp.max(w1, axis=1, keepdims=True)
    k, s1, s2 = stats(w1, m, z, z, z)

    def tk(_, carry):
        thresh, k, s1, s2 = carry
        w = s_ref[...]
        masked = jnp.where(w < thresh, w, -jnp.inf)
        m = jnp.max(masked, axis=1, keepdims=True)
        k, s1, s2 = stats(w, m, k, s1, s2)
        return m, k, s1, s2

    _, _, s1, s2 = jax.lax.fori_loop(0, _TOP_K - 1, tk, (m, k, s1, s2),
                                     unroll=True)
    rfin = r_ref[...]
    row_loss = (rfin * s2) / (rfin * s1 + _EPS)
    loss_b = jnp.sum(row_loss)

    @pl.when(b == 0)
    def _():
        out_ref[...] = jnp.zeros((1, 1), jnp.float32)

    out_ref[...] = out_ref[...] + loss_b * (1.0 / _B)


def _apml(pred, gt, interpret=False):
    predp = jnp.pad(pred, ((0, 0), (0, 0), (0, 5)))
    gttp = jnp.pad(jnp.swapaxes(gt, 1, 2), ((0, 0), (0, 5), (0, 0)))
    out = pl.pallas_call(
        _apml_kernel,
        grid=(_B,),
        in_specs=[
            pl.BlockSpec((1, _N, 8), lambda b: (b, 0, 0)),
            pl.BlockSpec((1, 8, _M), lambda b: (b, 0, 0)),
        ],
        out_specs=pl.BlockSpec((1, 1), lambda b: (0, 0)),
        out_shape=jax.ShapeDtypeStruct((1, 1), jnp.float32),
        scratch_shapes=[
            pltpu.VMEM((_N, _M), jnp.float32),
            pltpu.VMEM((_N, 1), jnp.float32),
        ],
        compiler_params=pltpu.CompilerParams(
            dimension_semantics=("arbitrary",),
        ),
        interpret=interpret,
    )(predp, gttp)
    return out[0, 0]


def kernel(pred, gt):
    return _apml(pred, gt)


# speedup vs baseline: 1.4188x; 1.1772x over previous
import jax
import jax.numpy as jnp
from jax.experimental import pallas as pl
from jax.experimental.pallas import tpu as pltpu

_TAU = 0.01
_SINKHORN_ITERS = 5
_EPS = 1e-05
_TOP_K = 5

_B, _N, _M = 8, 2048, 2048
_CHUNK = 256
_NCH = _N // _CHUNK


def _apml_kernel(pred_ref, gtt_ref, out_ref, s_ref, r_ref):
    b = pl.program_id(0)

    gtt = gtt_ref[0]
    b2 = jnp.sum(gtt * gtt, axis=0, keepdims=True)

    def build(i, colacc):
        sl = pl.ds(i * _CHUNK, _CHUNK)
        a = pred_ref[0, sl, :]
        a2 = jnp.sum(a * a, axis=1, keepdims=True)
        ab = jnp.dot(a, gtt, preferred_element_type=jnp.float32)
        d2 = jnp.maximum(a2 + b2 - 2.0 * ab, 1e-12)
        dchunk = d2 * jax.lax.rsqrt(d2)
        s = jnp.exp(dchunk * (-1.0 / _TAU))
        s_ref[sl, :] = s
        rs = jnp.sum(s, axis=1, keepdims=True)
        r_new = 1.0 / (rs + _EPS)
        r_ref[sl, :] = r_new
        return colacc + jnp.sum(s * r_new, axis=0, keepdims=True)

    cs = jax.lax.fori_loop(0, _NCH, build, jnp.zeros((1, _M), jnp.float32),
                           unroll=False)
    c0 = 1.0 / (cs + _EPS)

    def sink(_, c):
        rs = jnp.dot(s_ref[...], c.T, preferred_element_type=jnp.float32)
        r_old = r_ref[...]
        r_new = r_old / (r_old * rs + _EPS)
        r_ref[...] = r_new
        cs = jnp.dot(r_new.T, s_ref[...], preferred_element_type=jnp.float32)
        return c / (c * cs + _EPS)

    c = jax.lax.fori_loop(0, _SINKHORN_ITERS - 1, sink, c0, unroll=False)

    lnc = jnp.log(jnp.maximum(c, 1e-45))

    red_cols = jnp.concatenate([jnp.ones((_M, 1), jnp.float32), lnc.T],
                               axis=1)

    def stats(w, m, k, s1, s2):
        eqf = jnp.where(w == m, 1.0, 0.0)
        red = jnp.dot(eqf, red_cols, preferred_element_type=jnp.float32)
        cnt = red[:, 0:1]
        slnc = red[:, 1:2]
        live = (k < float(_TOP_K)) & (m > 0.0)
        sum_d = (cnt * jnp.log(m) - slnc) * (-_TAU)
        s1 = s1 + jnp.where(live, m * cnt, 0.0)
        s2 = s2 + jnp.where(live, m * sum_d, 0.0)
        k = k + jnp.where(k < float(_TOP_K), cnt, 0.0)
        return k, s1, s2

    z = jnp.zeros((_N, 1), jnp.float32)
    w1 = s_ref[...] * c
    s_ref[...] = w1
    m = jnp.max(w1, axis=1, keepdims=True)
    k, s1, s2 = stats(w1, m, z, z, z)

    def tk(_, carry):
        thresh, k, s1, s2 = carry
        w = s_ref[...]
        masked = jnp.where(w < thresh, w, -jnp.inf)
        m = jnp.max(masked, axis=1, keepdims=True)
        k, s1, s2 = stats(w, m, k, s1, s2)
        return m, k, s1, s2

    _, _, s1, s2 = jax.lax.fori_loop(0, _TOP_K - 1, tk, (m, k, s1, s2),
                                     unroll=True)
    rfin = r_ref[...]
    row_loss = (rfin * s2) / (rfin * s1 + _EPS)
    loss_b = jnp.sum(row_loss)

    @pl.when(b == 0)
    def _():
        out_ref[...] = jnp.zeros((1, 1), jnp.float32)

    out_ref[...] = out_ref[...] + loss_b * (1.0 / _B)


def _apml(pred, gt, interpret=False):
    predp = jnp.pad(pred, ((0, 0), (0, 0), (0, 5)))
    gttp = jnp.pad(jnp.swapaxes(gt, 1, 2), ((0, 0), (0, 5), (0, 0)))
    out = pl.pallas_call(
        _apml_kernel,
        grid=(_B,),
        in_specs=[
            pl.BlockSpec((1, _N, 8), lambda b: (b, 0, 0)),
            pl.BlockSpec((1, 8, _M), lambda b: (b, 0, 0)),
        ],
        out_specs=pl.BlockSpec((1, 1), lambda b: (0, 0)),
        out_shape=jax.ShapeDtypeStruct((1, 1), jnp.float32),
        scratch_shapes=[
            pltpu.VMEM((_N, _M), jnp.float32),
            pltpu.VMEM((_N, 1), jnp.float32),
        ],
        compiler_params=pltpu.CompilerParams(
            dimension_semantics=("arbitrary",),
        ),
        interpret=interpret,
    )(predp, gttp)
    return out[0, 0]


def kernel(pred, gt):
    return _apml(pred, gt)


# speedup vs baseline: 1.4206x; 1.0012x over previous
import jax
import jax.numpy as jnp
from jax.experimental import pallas as pl
from jax.experimental.pallas import tpu as pltpu

_TAU = 0.01
_SINKHORN_ITERS = 5
_EPS = 1e-05
_TOP_K = 5

_B, _N, _M = 8, 2048, 2048
_CHUNK = 256
_NCH = _N // _CHUNK


def _apml_kernel(pred_ref, gtt_ref, out_ref, s_ref, r_ref):
    b = pl.program_id(0)

    gtt = gtt_ref[0]
    b2 = jnp.sum(gtt * gtt, axis=0, keepdims=True)

    def build(i, colacc):
        sl = pl.ds(i * _CHUNK, _CHUNK)
        a = pred_ref[0, sl, :]
        a2 = jnp.sum(a * a, axis=1, keepdims=True)
        ab = jnp.dot(a, gtt, preferred_element_type=jnp.float32)
        d2 = jnp.maximum(a2 + b2 - 2.0 * ab, 1e-12)
        dchunk = d2 * jax.lax.rsqrt(d2)
        s = jnp.exp(dchunk * (-1.0 / _TAU))
        s_ref[sl, :] = s
        rs = jnp.sum(s, axis=1, keepdims=True)
        r_new = 1.0 / (rs + _EPS)
        r_ref[sl, :] = r_new
        return colacc + jnp.sum(s * r_new, axis=0, keepdims=True)

    cs = jax.lax.fori_loop(0, _NCH, build, jnp.zeros((1, _M), jnp.float32),
                           unroll=False)
    c0 = 1.0 / (cs + _EPS)

    def sink(_, c):
        rs = jnp.dot(s_ref[...], c.T, preferred_element_type=jnp.float32)
        r_old = r_ref[...]
        r_new = r_old / (r_old * rs + _EPS)
        r_ref[...] = r_new
        cs = jnp.dot(r_new.T, s_ref[...], preferred_element_type=jnp.float32)
        return c / (c * cs + _EPS)

    c = jax.lax.fori_loop(0, _SINKHORN_ITERS - 1, sink, c0, unroll=False)

    lnc = jnp.log(jnp.maximum(c, 1e-45))

    red_cols = jnp.concatenate([jnp.ones((_M, 1), jnp.float32), lnc.T],
                               axis=1)

    def stats(w, m, k, s1, s2):
        eqf = jnp.where(w == m, 1.0, 0.0)
        red = jnp.dot(eqf, red_cols, preferred_element_type=jnp.float32)
        cnt = red[:, 0:1]
        slnc = red[:, 1:2]
        live = (k < float(_TOP_K)) & (m > 0.0)
        sum_d = (cnt * jnp.log(m) - slnc) * (-_TAU)
        s1 = s1 + jnp.where(live, m * cnt, 0.0)
        s2 = s2 + jnp.where(live, m * sum_d, 0.0)
        k = k + jnp.where(k < float(_TOP_K), cnt, 0.0)
        return k, s1, s2

    z = jnp.zeros((_N, 1), jnp.float32)
    w1 = s_ref[...] * c
    s_ref[...] = w1
    m = jnp.max(w1, axis=1, keepdims=True)
    k, s1, s2 = stats(w1, m, z, z, z)

    def tk(_, carry):
        thresh, k, s1, s2 = carry
        w = s_ref[...]
        masked = jnp.where(w < thresh, w, -jnp.inf)
        m = jnp.max(masked, axis=1, keepdims=True)
        k, s1, s2 = stats(w, m, k, s1, s2)
        return m, k, s1, s2

    _, _, s1, s2 = jax.lax.fori_loop(0, _TOP_K - 1, tk, (m, k, s1, s2),
                                     unroll=True)
    rfin = r_ref[...]
    row_loss = (rfin * s2) / (rfin * s1 + _EPS)
    loss_b = jnp.sum(row_loss)

    @pl.when(b == 0)
    def _():
        out_ref[...] = jnp.zeros((1, 1), jnp.float32)

    out_ref[...] = out_ref[...] + loss_b * (1.0 / _B)


def _apml(pred, gt):
    predp = jnp.pad(pred, ((0, 0), (0, 0), (0, 5)))
    gttp = jnp.pad(jnp.swapaxes(gt, 1, 2), ((0, 0), (0, 5), (0, 0)))
    out = pl.pallas_call(
        _apml_kernel,
        grid=(_B,),
        in_specs=[
            pl.BlockSpec((1, _N, 8), lambda b: (b, 0, 0)),
            pl.BlockSpec((1, 8, _M), lambda b: (b, 0, 0)),
        ],
        out_specs=pl.BlockSpec((1, 1), lambda b: (0, 0)),
        out_shape=jax.ShapeDtypeStruct((1, 1), jnp.float32),
        scratch_shapes=[
            pltpu.VMEM((_N, _M), jnp.float32),
            pltpu.VMEM((_N, 1), jnp.float32),
        ],
        compiler_params=pltpu.CompilerParams(
            dimension_semantics=("arbitrary",),
        ),
    )(predp, gttp)
    return out[0, 0]


def kernel(pred, gt):
    return _apml(pred, gt)
